# baseline (speedup 1.0000x reference)
import functools

import jax
import jax.numpy as jnp
from jax import lax
from jax.experimental import pallas as pl
from jax.experimental.pallas import tpu as pltpu
from jax.experimental.pallas import tpu_sc as plsc

L = 16384
DIM = 64
VOCAB = 1000000
HIST = 1001472
NC = 2
NS = 16
NW = NC * NS
PER_TILE = L // NW
SCHUNK = 128
NSCHUNK = PER_TILE // SCHUNK
ZBUF = 7824
ZREP = HIST // NS // ZBUF
assert ZBUF % 16 == 0 and ZREP * ZBUF * NS == HIST

TCBLK = 16384
TC_NBLK = 44
SC_COL0 = TC_NBLK * TCBLK
SCBLK = 128
SC_NBLK = (VOCAB - SC_COL0 - 64) // SCBLK
TAIL0 = SC_COL0 + SC_NBLK * SCBLK
NITER = -(-SC_NBLK // NS)


def _sc_hist_and_scan(token_ids, tab_t):
    mesh = plsc.VectorSubcoreMesh(core_axis_name="c", subcore_axis_name="s")

    @functools.partial(
        pl.kernel,
        mesh=mesh,
        out_type=(jax.ShapeDtypeStruct((HIST,), jnp.float32),
                  jax.ShapeDtypeStruct((HIST,), jnp.float32),
                  jax.ShapeDtypeStruct((2 * DIM * 16,), jnp.float32)),
        scratch_types=[
            pltpu.VMEM((NSCHUNK, SCHUNK), jnp.int32),
            pltpu.VMEM((SCHUNK,), jnp.float32),
            pltpu.VMEM((ZBUF,), jnp.float32),
            pltpu.VMEM((2, DIM, SCBLK), jnp.float32),
            pltpu.VMEM((2, SCBLK), jnp.float32),
            pltpu.VMEM((DIM * 16,), jnp.float32),
            pltpu.VMEM((DIM * 16,), jnp.float32),
            pltpu.VMEM((NS * DIM * 16,), jnp.float32),
            pltpu.VMEM_SHARED((HIST,), jnp.float32),
            pltpu.VMEM_SHARED((NS * DIM * 16,), jnp.float32),
            pltpu.SemaphoreType.DMA,
            pltpu.SemaphoreType.DMA((2,)),
            pltpu.SemaphoreType.DMA((2,)),
        ],
    )
    def k(tok_hbm, tab_hbm, out0_hbm, out1_hbm, outp_hbm,
          idx_v, ones_v, zbuf_v, sbuf_v, scnt_v, accw_v, fold_v, sum_v,
          hist_sh, seg_sh, zsem, tsem, csem):
        c = lax.axis_index("c")
        s = lax.axis_index("s")
        wid = s * NC + c
        base = wid * PER_TILE

        for q in range(NSCHUNK):
            pltpu.sync_copy(tok_hbm.at[pl.ds(base + q * SCHUNK, SCHUNK)],
                            idx_v.at[q])

        one16 = jnp.full((16,), 1.0, jnp.float32)
        for i in range(SCHUNK // 16):
            ones_v[pl.ds(i * 16, 16)] = one16

        z16 = jnp.zeros((16,), jnp.float32)

        def zfill(i, _):
            zbuf_v[pl.ds(i * 16, 16)] = z16
            return 0

        lax.fori_loop(0, ZBUF // 16, zfill, 0)

        zdescs = [
            pltpu.async_copy(
                zbuf_v,
                hist_sh.at[pl.ds((s * ZREP + r) * ZBUF, ZBUF)],
                zsem)
            for r in range(ZREP)
        ]
        for d in zdescs:
            d.wait()

        def zacc(i, _):
            accw_v[pl.ds(i * 16, 16)] = z16
            return 0

        lax.fori_loop(0, DIM, zacc, 0)

        plsc.subcore_barrier()

        for q in range(NSCHUNK):
            pltpu.sync_copy(ones_v, hist_sh.at[idx_v.at[q]], add=True)

        plsc.subcore_barrier()

        @pl.when(jnp.logical_and(s == 0, c == 0))
        def _emit0():
            pltpu.sync_copy(hist_sh, out0_hbm)

        @pl.when(jnp.logical_and(s == 0, c == 1))
        def _emit1():
            pltpu.sync_copy(hist_sh, out1_hbm)

        def col_of(i):
            return pl.multiple_of(SC_COL0 + (i * NS + s) * SCBLK, SCBLK)

        def fire(i, par):
            col = col_of(i)

            @pl.when(i * NS + s < SC_NBLK)
            def _():
                pltpu.async_copy(tab_hbm.at[:, pl.ds(col, SCBLK)],
                                 sbuf_v.at[par], tsem.at[par])
                pltpu.async_copy(hist_sh.at[pl.ds(col, SCBLK)],
                                 scnt_v.at[par], csem.at[par])

        def process(i, par):
            @pl.when(i * NS + s < SC_NBLK)
            def _():
                pltpu.make_async_copy(tab_hbm.at[:, pl.ds(0, SCBLK)],
                                      sbuf_v.at[par], tsem.at[par]).wait()
                pltpu.make_async_copy(hist_sh.at[pl.ds(0, SCBLK)],
                                      scnt_v.at[par], csem.at[par]).wait()
                cnt = [scnt_v[par, pl.ds(k * 16, 16)] for k in range(8)]

                def dloop(d8, _):
                    for dd in range(8):
                        d = d8 * 8 + dd
                        a = accw_v[pl.ds(d * 16, 16)]
                        for k in range(8):
                            a = a + sbuf_v[par, d, pl.ds(k * 16, 16)] * cnt[k]
                        accw_v[pl.ds(d * 16, 16)] = a
                    return 0

                lax.fori_loop(0, DIM // 8, dloop, 0)

        fire(0, 0)

        def outer(i2, _):
            i0 = i2 * 2
            fire(i0 + 1, 1)
            process(i0, 0)
            fire(i0 + 2, 0)
            process(i0 + 1, 1)
            return 0

        lax.fori_loop(0, -(-NITER // 2), outer, 0)

        pltpu.sync_copy(accw_v, seg_sh.at[pl.ds(s * DIM * 16, DIM * 16)])

        plsc.subcore_barrier()

        @pl.when(s == 0)
        def _emitp():
            pltpu.sync_copy(seg_sh, sum_v)

            def dsum(d, _):
                tot = sum_v[pl.ds(d * 16, 16)]
                for ss in range(1, NS):
                    tot = tot + sum_v[pl.ds(ss * DIM * 16 + d * 16, 16)]
                fold_v[pl.ds(d * 16, 16)] = tot
                return 0

            lax.fori_loop(0, DIM, dsum, 0)
            pltpu.sync_copy(fold_v, outp_hbm.at[pl.ds(c * DIM * 16,
                                                      DIM * 16)])

    return k(token_ids, tab_t)


def _scan_body(tab_ref, c0_ref, c1_ref, scp_ref, fold_ref, tabt_ref,
               cntt_ref, w_ref, o_ref, acc_ref):
    i = pl.program_id(0)

    @pl.when(i == 0)
    def _init():
        acc_ref[...] = jnp.zeros_like(acc_ref)

    cnt = c0_ref[...] + c1_ref[...]
    contrib = lax.dot_general(
        cnt, tab_ref[...], (((1,), (1,)), ((), ())),
        preferred_element_type=jnp.float32)
    acc_ref[...] += contrib

    @pl.when(i == TC_NBLK - 1)
    def _fin():
        sc_tot = lax.dot_general(
            scp_ref[...], fold_ref[...], (((1,), (0,)), ((), ())),
            preferred_element_type=jnp.float32)
        tail = lax.dot_general(
            cntt_ref[...], tabt_ref[...], (((1,), (1,)), ((), ())),
            preferred_element_type=jnp.float32)
        total = acc_ref[...] + tail + sc_tot
        pooled = total * (1.0 / L)
        o_ref[...] = lax.dot_general(
            pooled, w_ref[...], (((1,), (1,)), ((), ())),
            preferred_element_type=jnp.float32)


def kernel(token_ids, table, W):
    tab_t = table.T
    c0, c1, scp = _sc_hist_and_scan(token_ids, tab_t)
    tab_tail = lax.slice(tab_t, (0, TAIL0), (DIM, VOCAB))
    cnt_tail = (c0[TAIL0:VOCAB] + c1[TAIL0:VOCAB]).reshape(1, 64)
    fold_m = jnp.equal(
        (jnp.arange(2 * DIM * 16)[:, None] % (DIM * 16)) // 16,
        jnp.arange(DIM)[None, :]).astype(jnp.float32)
    return pl.pallas_call(
        _scan_body,
        grid=(TC_NBLK,),
        in_specs=[
            pl.BlockSpec((DIM, TCBLK), lambda i: (0, i)),
            pl.BlockSpec((1, TCBLK), lambda i: (0, i)),
            pl.BlockSpec((1, TCBLK), lambda i: (0, i)),
            pl.BlockSpec((1, 2 * DIM * 16), lambda i: (0, 0)),
            pl.BlockSpec((2 * DIM * 16, DIM), lambda i: (0, 0)),
            pl.BlockSpec((DIM, 64), lambda i: (0, 0)),
            pl.BlockSpec((1, 64), lambda i: (0, 0)),
            pl.BlockSpec((DIM, DIM), lambda i: (0, 0)),
        ],
        out_specs=pl.BlockSpec((1, DIM), lambda i: (0, 0)),
        out_shape=jax.ShapeDtypeStruct((1, DIM), jnp.float32),
        scratch_shapes=[pltpu.VMEM((1, DIM), jnp.float32)],
    )(tab_t, c0.reshape(1, HIST), c1.reshape(1, HIST),
      scp.reshape(1, 2 * DIM * 16), fold_m, tab_tail, cnt_tail, W)


# speedup vs baseline: 1.8702x; 1.8702x over previous
import functools

import jax
import jax.numpy as jnp
from jax import lax
from jax.experimental import pallas as pl
from jax.experimental.pallas import tpu as pltpu
from jax.experimental.pallas import tpu_sc as plsc

L = 16384
DIM = 64
VOCAB = 1000000
HIST = 1 << 20
NC = 2
NS = 16
NW = NC * NS
PER_TILE = L // NW
SCHUNK = 128
NSCHUNK = PER_TILE // SCHUNK
ZBUF = 8192
ZREP = HIST // NS // ZBUF

BLK = 16384
NBLK = -(-VOCAB // BLK)


def _sc_histograms(token_ids):
    mesh = plsc.VectorSubcoreMesh(core_axis_name="c", subcore_axis_name="s")

    @functools.partial(
        pl.kernel,
        mesh=mesh,
        out_type=(jax.ShapeDtypeStruct((HIST,), jnp.float32),
                  jax.ShapeDtypeStruct((HIST,), jnp.float32)),
        scratch_types=[
            pltpu.VMEM((NSCHUNK, SCHUNK), jnp.int32),
            pltpu.VMEM((SCHUNK,), jnp.float32),
            pltpu.VMEM((ZBUF,), jnp.float32),
            pltpu.VMEM_SHARED((HIST,), jnp.float32),
            pltpu.SemaphoreType.DMA,
        ],
    )
    def k(tok_hbm, out0_hbm, out1_hbm, idx_v, ones_v, zbuf_v, hist_sh, zsem):
        c = lax.axis_index("c")
        s = lax.axis_index("s")
        wid = s * NC + c
        base = wid * PER_TILE

        for q in range(NSCHUNK):
            pltpu.sync_copy(tok_hbm.at[pl.ds(base + q * SCHUNK, SCHUNK)],
                            idx_v.at[q])

        one16 = jnp.full((16,), 1.0, jnp.float32)
        for i in range(SCHUNK // 16):
            ones_v[pl.ds(i * 16, 16)] = one16

        z16 = jnp.zeros((16,), jnp.float32)

        def zfill(i, _):
            zbuf_v[pl.ds(i * 16, 16)] = z16
            return 0

        lax.fori_loop(0, ZBUF // 16, zfill, 0)

        zdescs = [
            pltpu.async_copy(
                zbuf_v,
                hist_sh.at[pl.ds((s * ZREP + r) * ZBUF, ZBUF)],
                zsem)
            for r in range(ZREP)
        ]
        for d in zdescs:
            d.wait()

        plsc.subcore_barrier()

        for q in range(NSCHUNK):
            pltpu.sync_copy(ones_v, hist_sh.at[idx_v.at[q]], add=True)

        plsc.subcore_barrier()

        @pl.when(jnp.logical_and(s == 0, c == 0))
        def _emit0():
            pltpu.sync_copy(hist_sh, out0_hbm)

        @pl.when(jnp.logical_and(s == 0, c == 1))
        def _emit1():
            pltpu.sync_copy(hist_sh, out1_hbm)

    return k(token_ids)


def _scan_body(tab_ref, c0_ref, c1_ref, w_ref, o_ref, acc_ref):
    i = pl.program_id(0)

    @pl.when(i == 0)
    def _init():
        acc_ref[...] = jnp.zeros_like(acc_ref)

    cnt = c0_ref[...] + c1_ref[...]
    contrib = lax.dot_general(
        cnt, tab_ref[...], (((1,), (1,)), ((), ())),
        preferred_element_type=jnp.float32)
    acc_ref[...] += contrib

    @pl.when(i == NBLK - 1)
    def _fin():
        pooled = acc_ref[...] * (1.0 / L)
        o_ref[...] = lax.dot_general(
            pooled, w_ref[...], (((1,), (1,)), ((), ())),
            preferred_element_type=jnp.float32)


def kernel(token_ids, table, W):
    c0, c1 = _sc_histograms(token_ids)
    tab_t = table.T
    return pl.pallas_call(
        _scan_body,
        grid=(NBLK,),
        in_specs=[
            pl.BlockSpec((DIM, BLK), lambda i: (0, i)),
            pl.BlockSpec((1, BLK), lambda i: (0, i)),
            pl.BlockSpec((1, BLK), lambda i: (0, i)),
            pl.BlockSpec((DIM, DIM), lambda i: (0, 0)),
        ],
        out_specs=pl.BlockSpec((1, DIM), lambda i: (0, 0)),
        out_shape=jax.ShapeDtypeStruct((1, DIM), jnp.float32),
        scratch_shapes=[pltpu.VMEM((1, DIM), jnp.float32)],
    )(tab_t, c0.reshape(1, HIST), c1.reshape(1, HIST), W)


# speedup vs baseline: 2.1243x; 1.1359x over previous
import functools

import jax
import jax.numpy as jnp
from jax import lax
from jax.experimental import pallas as pl
from jax.experimental.pallas import tpu as pltpu
from jax.experimental.pallas import tpu_sc as plsc

L = 16384
DIM = 64
VOCAB = 1000000
HIST = 1 << 20
NC = 2
NS = 16
NW = NC * NS
PER_TILE = L // NW
SCHUNK = 128
NSCHUNK = PER_TILE // SCHUNK
ZBUF = 8192
ZREP = HIST // NS // ZBUF

BLK = 32768
NBLK = -(-VOCAB // BLK)


def _sc_histograms(token_ids):
    mesh = plsc.VectorSubcoreMesh(core_axis_name="c", subcore_axis_name="s")

    @functools.partial(
        pl.kernel,
        mesh=mesh,
        out_type=(jax.ShapeDtypeStruct((HIST,), jnp.float32),
                  jax.ShapeDtypeStruct((HIST,), jnp.float32)),
        scratch_types=[
            pltpu.VMEM((NSCHUNK, SCHUNK), jnp.int32),
            pltpu.VMEM((SCHUNK,), jnp.float32),
            pltpu.VMEM((ZBUF,), jnp.float32),
            pltpu.VMEM_SHARED((HIST,), jnp.float32),
            pltpu.SemaphoreType.DMA,
        ],
    )
    def k(tok_hbm, out0_hbm, out1_hbm, idx_v, ones_v, zbuf_v, hist_sh, zsem):
        c = lax.axis_index("c")
        s = lax.axis_index("s")
        wid = s * NC + c
        base = wid * PER_TILE

        for q in range(NSCHUNK):
            pltpu.sync_copy(tok_hbm.at[pl.ds(base + q * SCHUNK, SCHUNK)],
                            idx_v.at[q])

        one16 = jnp.full((16,), 1.0, jnp.float32)
        for i in range(SCHUNK // 16):
            ones_v[pl.ds(i * 16, 16)] = one16

        z16 = jnp.zeros((16,), jnp.float32)

        def zfill(i, _):
            zbuf_v[pl.ds(i * 16, 16)] = z16
            return 0

        lax.fori_loop(0, ZBUF // 16, zfill, 0)

        zdescs = [
            pltpu.async_copy(
                zbuf_v,
                hist_sh.at[pl.ds((s * ZREP + r) * ZBUF, ZBUF)],
                zsem)
            for r in range(ZREP)
        ]
        for d in zdescs:
            d.wait()

        plsc.subcore_barrier()

        for q in range(NSCHUNK):
            pltpu.sync_copy(ones_v, hist_sh.at[idx_v.at[q]], add=True)

        plsc.subcore_barrier()

        @pl.when(jnp.logical_and(s == 0, c == 0))
        def _emit0():
            pltpu.sync_copy(hist_sh, out0_hbm)

        @pl.when(jnp.logical_and(s == 0, c == 1))
        def _emit1():
            pltpu.sync_copy(hist_sh, out1_hbm)

    return k(token_ids)


def _scan_body(tab_ref, c0_ref, c1_ref, w_ref, o_ref, acc_ref):
    i = pl.program_id(0)

    @pl.when(i == 0)
    def _init():
        acc_ref[...] = jnp.zeros_like(acc_ref)

    cnt = c0_ref[...] + c1_ref[...]
    contrib = lax.dot_general(
        cnt, tab_ref[...], (((1,), (1,)), ((), ())),
        preferred_element_type=jnp.float32)
    acc_ref[...] += contrib

    @pl.when(i == NBLK - 1)
    def _fin():
        pooled = acc_ref[...] * (1.0 / L)
        o_ref[...] = lax.dot_general(
            pooled, w_ref[...], (((1,), (1,)), ((), ())),
            preferred_element_type=jnp.float32)


def kernel(token_ids, table, W):
    c0, c1 = _sc_histograms(token_ids)
    tab_t = table.T
    return pl.pallas_call(
        _scan_body,
        grid=(NBLK,),
        in_specs=[
            pl.BlockSpec((DIM, BLK), lambda i: (0, i)),
            pl.BlockSpec((1, BLK), lambda i: (0, i)),
            pl.BlockSpec((1, BLK), lambda i: (0, i)),
            pl.BlockSpec((DIM, DIM), lambda i: (0, 0)),
        ],
        out_specs=pl.BlockSpec((1, DIM), lambda i: (0, 0)),
        out_shape=jax.ShapeDtypeStruct((1, DIM), jnp.float32),
        scratch_shapes=[pltpu.VMEM((1, DIM), jnp.float32)],
    )(tab_t, c0.reshape(1, HIST), c1.reshape(1, HIST), W)
